# baseline (speedup 1.0000x reference)
import functools

import jax
import jax.numpy as jnp
from jax import lax
from jax.experimental import pallas as pl
from jax.experimental.pallas import tpu as pltpu
from jax.experimental.pallas import tpu_sc as plsc

_N = 10000
_NPAD = 10240
_E = 320000
_DIN = 128
_H = 64
_NC = 2
_NS = 16
_NW = _NC * _NS
_CHUNK = 128
_KCH = 81
_NG = _KCH // 3
_EPW = _CHUNK * _KCH
_EPAD = _EPW * _NW
_RPT = _NPAD // _NS
_BLK = 1024
_GRID = _NPAD // _BLK

_sc_mesh = plsc.VectorSubcoreMesh(core_axis_name="c", subcore_axis_name="s")


def _deg_body(dst_hbm, zeros_hbm, out_hbm, dst_v, ones_v, accum_sh, dsems):
    cid = lax.axis_index("c")
    sid = lax.axis_index("s")
    wid = cid * _NS + sid

    def fill(i, c):
        ones_v[i] = jnp.full((16,), 1.0, jnp.float32)
        return c
    lax.fori_loop(0, _CHUNK, fill, 0)

    pre = [
        pltpu.async_copy(zeros_hbm.at[pl.ds(sid * _RPT, _RPT), pl.ds(0, 16)],
                         accum_sh.at[pl.ds(sid * _RPT, _RPT)], dsems[0]),
        pltpu.async_copy(dst_hbm.at[wid], dst_v, dsems[1]),
    ]
    for d in pre:
        d.wait()
    plsc.subcore_barrier()

    def body(j, c):
        pltpu.sync_copy(ones_v, accum_sh.at[dst_v.at[j]], add=True)
        return c
    lax.fori_loop(0, _KCH, body, 0)

    plsc.subcore_barrier()
    pltpu.sync_copy(accum_sh.at[pl.ds(sid * _RPT, _RPT)],
                    out_hbm.at[cid, pl.ds(sid * _RPT, _RPT)])


_deg_call = pl.kernel(
    _deg_body,
    out_type=jax.ShapeDtypeStruct((_NC, _NPAD, 16), jnp.float32),
    mesh=_sc_mesh,
    compiler_params=pltpu.CompilerParams(use_tc_tiling_on_sc=False, skip_device_barrier=True, disable_bounds_checks=True),
    scratch_types=[
        pltpu.VMEM((_KCH, _CHUNK), jnp.int32),
        pltpu.VMEM((_CHUNK, 16), jnp.float32),
        pltpu.VMEM_SHARED((_NPAD, 16), jnp.float32),
        [pltpu.SemaphoreType.DMA] * 2,
    ],
)


def _agg_body(y_hbm, src_hbm, dst_hbm, zeros_hbm, out_hbm,
              src_v, dst_v, rows_v, rows_w, rows_u, y_sh, accum_sh, gsems,
              psems):
    cid = lax.axis_index("c")
    sid = lax.axis_index("s")
    wid = cid * _NS + sid

    pre = [
        pltpu.async_copy(zeros_hbm.at[pl.ds(sid * _RPT, _RPT)],
                         accum_sh.at[pl.ds(sid * _RPT, _RPT)], psems[0]),
        pltpu.async_copy(y_hbm.at[pl.ds(sid * _RPT, _RPT)],
                         y_sh.at[pl.ds(sid * _RPT, _RPT)], psems[1]),
        pltpu.async_copy(src_hbm.at[wid], src_v, psems[2]),
        pltpu.async_copy(dst_hbm.at[wid], dst_v, psems[3]),
    ]
    for d in pre:
        d.wait()
    plsc.subcore_barrier()

    bufs = [rows_v, rows_w, rows_u]

    def start_gather(buf, gs, j):
        pltpu.async_copy(y_sh.at[src_v.at[j]], buf, gs)

    def wait_gather(buf, gs):
        pltpu.make_async_copy(y_sh.at[src_v.at[0]], buf, gs).wait()

    for b in range(3):
        start_gather(bufs[b], gsems[b], b)

    def trio(i, c):
        base = i * 3
        for b in range(3):
            wait_gather(bufs[b], gsems[b])
            pltpu.sync_copy(bufs[b], accum_sh.at[dst_v.at[base + b]],
                            add=True)

            @pl.when(i < _NG - 1)
            def _():
                start_gather(bufs[b], gsems[b], base + 3 + b)
        return c
    lax.fori_loop(0, _NG, trio, 0)

    plsc.subcore_barrier()
    pltpu.sync_copy(accum_sh.at[pl.ds(sid * _RPT, _RPT)],
                    out_hbm.at[cid, pl.ds(sid * _RPT, _RPT)])


_agg_call = pl.kernel(
    _agg_body,
    out_type=jax.ShapeDtypeStruct((_NC, _NPAD, _H), jnp.float32),
    mesh=_sc_mesh,
    compiler_params=pltpu.CompilerParams(use_tc_tiling_on_sc=False, skip_device_barrier=True, disable_bounds_checks=True),
    scratch_types=[
        pltpu.VMEM((_KCH, _CHUNK), jnp.int32),
        pltpu.VMEM((_KCH, _CHUNK), jnp.int32),
        pltpu.VMEM((_CHUNK, _H), jnp.float32),
        pltpu.VMEM((_CHUNK, _H), jnp.float32),
        pltpu.VMEM((_CHUNK, _H), jnp.float32),
        pltpu.VMEM_SHARED((_NPAD, _H), jnp.float32),
        pltpu.VMEM_SHARED((_NPAD, _H), jnp.float32),
        [pltpu.SemaphoreType.DMA] * 3,
        [pltpu.SemaphoreType.DMA] * 4,
    ],
)


def _tc1_body(p_ref, x_ref, w1_ref, y_ref, dinv_ref):
    deg = p_ref[0, :, 0:1] + p_ref[1, :, 0:1] + 1.0
    dinv = lax.rsqrt(deg)
    xw = jnp.dot(x_ref[:], w1_ref[:], preferred_element_type=jnp.float32)
    y_ref[:] = dinv * xw
    dinv_ref[:] = dinv


_tc1_call = pl.pallas_call(
    _tc1_body,
    grid=(_GRID,),
    in_specs=[
        pl.BlockSpec((_NC, _BLK, 16), lambda i: (0, i, 0)),
        pl.BlockSpec((_BLK, _DIN), lambda i: (i, 0)),
        pl.BlockSpec((_DIN, _H), lambda i: (0, 0)),
    ],
    out_specs=[
        pl.BlockSpec((_BLK, _H), lambda i: (i, 0)),
        pl.BlockSpec((_BLK, 1), lambda i: (i, 0)),
    ],
    out_shape=[
        jax.ShapeDtypeStruct((_NPAD, _H), jnp.float32),
        jax.ShapeDtypeStruct((_NPAD, 1), jnp.float32),
    ],
)


def _tc2_body(p_ref, y1_ref, dinv_ref, b1_ref, w2_ref, y2_ref):
    s = p_ref[0] + p_ref[1] + y1_ref[:]
    dinv = dinv_ref[:]
    g1 = jnp.maximum(dinv * s + b1_ref[:], 0.0)
    y2_ref[:] = dinv * jnp.dot(g1, w2_ref[:], preferred_element_type=jnp.float32)


_tc2_call = pl.pallas_call(
    _tc2_body,
    grid=(_GRID,),
    in_specs=[
        pl.BlockSpec((_NC, _BLK, _H), lambda i: (0, i, 0)),
        pl.BlockSpec((_BLK, _H), lambda i: (i, 0)),
        pl.BlockSpec((_BLK, 1), lambda i: (i, 0)),
        pl.BlockSpec((1, _H), lambda i: (0, 0)),
        pl.BlockSpec((_H, _H), lambda i: (0, 0)),
    ],
    out_specs=pl.BlockSpec((_BLK, _H), lambda i: (i, 0)),
    out_shape=jax.ShapeDtypeStruct((_NPAD, _H), jnp.float32),
)


def _tc3_body(p_ref, y2_ref, dinv_ref, b2_ref, wd1_ref, bd1_ref, wd2_ref,
              bd2_ref, res_ref):
    s = p_ref[0] + p_ref[1] + y2_ref[:]
    dinv = dinv_ref[:]
    g2 = jnp.maximum(dinv * s + b2_ref[:], 0.0)
    d3 = jnp.maximum(
        jnp.dot(g2, wd1_ref[:], preferred_element_type=jnp.float32) + bd1_ref[:],
        0.0)
    res_ref[:] = jnp.dot(d3, wd2_ref[:],
                         preferred_element_type=jnp.float32) + bd2_ref[:]


_tc3_call = pl.pallas_call(
    _tc3_body,
    grid=(_GRID,),
    in_specs=[
        pl.BlockSpec((_NC, _BLK, _H), lambda i: (0, i, 0)),
        pl.BlockSpec((_BLK, _H), lambda i: (i, 0)),
        pl.BlockSpec((_BLK, 1), lambda i: (i, 0)),
        pl.BlockSpec((1, _H), lambda i: (0, 0)),
        pl.BlockSpec((_H, 32), lambda i: (0, 0)),
        pl.BlockSpec((1, 32), lambda i: (0, 0)),
        pl.BlockSpec((32, 1), lambda i: (0, 0)),
        pl.BlockSpec((1, 1), lambda i: (0, 0)),
    ],
    out_specs=pl.BlockSpec((_BLK, 1), lambda i: (i, 0)),
    out_shape=jax.ShapeDtypeStruct((_NPAD, 1), jnp.float32),
)


@jax.jit
def kernel(x, edge_index, W1, b1, W2, b2, Wd1, bd1, Wd2, bd2):
    src = edge_index[0]
    dst = edge_index[1]
    pad_idx = jnp.full((_EPAD - _E,), _N, jnp.int32)
    src_p = jnp.concatenate([src, pad_idx]).reshape(_NW, _KCH, _CHUNK)
    dst_p = jnp.concatenate([dst, pad_idx]).reshape(_NW, _KCH, _CHUNK)
    zeros64 = jnp.zeros((_NPAD, _H), jnp.float32)

    degp = _deg_call(dst_p, zeros64)
    y1, dinv = _tc1_call(degp, x, W1)
    s1 = _agg_call(y1, src_p, dst_p, zeros64)
    y2 = _tc2_call(s1, y1, dinv, b1.reshape(1, _H), W2)
    s2 = _agg_call(y2, src_p, dst_p, zeros64)
    res = _tc3_call(s2, y2, dinv, b2.reshape(1, _H), Wd1, bd1.reshape(1, 32),
                    Wd2, bd2.reshape(1, 1))
    return res[:_N]


# speedup vs baseline: 1.0448x; 1.0448x over previous
import functools

import jax
import jax.numpy as jnp
from jax import lax
from jax.experimental import pallas as pl
from jax.experimental.pallas import tpu as pltpu
from jax.experimental.pallas import tpu_sc as plsc

_N = 10000
_NPAD = 10240
_E = 320000
_DIN = 128
_H = 64
_NC = 2
_NS = 16
_NW = _NC * _NS
_CHUNK = 128
_KCH = 81
_NG = _KCH // 3
_EPW = _CHUNK * _KCH
_EPAD = _EPW * _NW
_RPT = _NPAD // _NS
_BLK = 1024
_GRID = _NPAD // _BLK

_sc_mesh = plsc.VectorSubcoreMesh(core_axis_name="c", subcore_axis_name="s")


_NVEC = _EPW // 16


def _deg_body(dst_hbm, out_hbm, dst_v, hist_v, red_v, out_v, part_sh, dsem):
    cid = lax.axis_index("c")
    sid = lax.axis_index("s")
    wid = cid * _NS + sid

    d = pltpu.async_copy(dst_hbm.at[wid], dst_v, dsem)

    def zero(i, c):
        hist_v[pl.ds(i * 16, 16)] = jnp.zeros((16,), jnp.float32)
        return c
    lax.fori_loop(0, _NPAD // 16, zero, 0)
    d.wait()

    ones = jnp.full((16,), 1.0, jnp.float32)

    def body(i, c):
        plsc.addupdate_scatter(hist_v, [dst_v[i]], ones)
        return c
    lax.fori_loop(0, _NVEC, body, 0)

    pltpu.sync_copy(hist_v, part_sh.at[sid])
    plsc.subcore_barrier()
    pltpu.sync_copy(part_sh.at[:, pl.ds(sid * _RPT, _RPT)], red_v)

    def red(i, c):
        acc = jnp.zeros((16,), jnp.float32)
        for r in range(_NS):
            acc = acc + red_v[r, pl.ds(i * 16, 16)]
        out_v[pl.ds(i * 16, 16)] = acc
        return c
    lax.fori_loop(0, _RPT // 16, red, 0)

    pltpu.sync_copy(out_v, out_hbm.at[cid, pl.ds(sid * _RPT, _RPT)])


_deg_call = pl.kernel(
    _deg_body,
    out_type=jax.ShapeDtypeStruct((_NC, _NPAD), jnp.float32),
    mesh=_sc_mesh,
    compiler_params=pltpu.CompilerParams(use_tc_tiling_on_sc=False,
                                         needs_layout_passes=False),
    scratch_types=[
        pltpu.VMEM((_NVEC, 16), jnp.int32),
        pltpu.VMEM((_NPAD,), jnp.float32),
        pltpu.VMEM((_NS, _RPT), jnp.float32),
        pltpu.VMEM((_RPT,), jnp.float32),
        pltpu.VMEM_SHARED((_NS, _NPAD), jnp.float32),
        pltpu.SemaphoreType.DMA,
    ],
)


def _agg_body(y_hbm, src_hbm, dst_hbm, zeros_hbm, out_hbm,
              src_v, dst_v, rows_v, rows_w, rows_u, y_sh, accum_sh, gsems,
              psems):
    cid = lax.axis_index("c")
    sid = lax.axis_index("s")
    wid = cid * _NS + sid

    pre = [
        pltpu.async_copy(zeros_hbm.at[pl.ds(sid * _RPT, _RPT)],
                         accum_sh.at[pl.ds(sid * _RPT, _RPT)], psems[0]),
        pltpu.async_copy(y_hbm.at[pl.ds(sid * _RPT, _RPT)],
                         y_sh.at[pl.ds(sid * _RPT, _RPT)], psems[1]),
        pltpu.async_copy(src_hbm.at[wid], src_v, psems[2]),
        pltpu.async_copy(dst_hbm.at[wid], dst_v, psems[3]),
    ]
    for d in pre:
        d.wait()
    plsc.subcore_barrier()

    bufs = [rows_v, rows_w, rows_u]

    def start_gather(buf, gs, j):
        pltpu.async_copy(y_sh.at[src_v.at[j]], buf, gs)

    def wait_gather(buf, gs):
        pltpu.make_async_copy(y_sh.at[src_v.at[0]], buf, gs).wait()

    for b in range(3):
        start_gather(bufs[b], gsems[b], b)

    def trio(i, c):
        base = i * 3
        for b in range(3):
            wait_gather(bufs[b], gsems[b])
            pltpu.sync_copy(bufs[b], accum_sh.at[dst_v.at[base + b]],
                            add=True)

            @pl.when(i < _NG - 1)
            def _():
                start_gather(bufs[b], gsems[b], base + 3 + b)
        return c
    lax.fori_loop(0, _NG, trio, 0)

    plsc.subcore_barrier()
    pltpu.sync_copy(accum_sh.at[pl.ds(sid * _RPT, _RPT)],
                    out_hbm.at[cid, pl.ds(sid * _RPT, _RPT)])


_agg_call = pl.kernel(
    _agg_body,
    out_type=jax.ShapeDtypeStruct((_NC, _NPAD, _H), jnp.float32),
    mesh=_sc_mesh,
    compiler_params=pltpu.CompilerParams(use_tc_tiling_on_sc=False),
    scratch_types=[
        pltpu.VMEM((_KCH, _CHUNK), jnp.int32),
        pltpu.VMEM((_KCH, _CHUNK), jnp.int32),
        pltpu.VMEM((_CHUNK, _H), jnp.float32),
        pltpu.VMEM((_CHUNK, _H), jnp.float32),
        pltpu.VMEM((_CHUNK, _H), jnp.float32),
        pltpu.VMEM_SHARED((_NPAD, _H), jnp.float32),
        pltpu.VMEM_SHARED((_NPAD, _H), jnp.float32),
        [pltpu.SemaphoreType.DMA] * 3,
        [pltpu.SemaphoreType.DMA] * 4,
    ],
)


def _tc1_body(p_ref, x_ref, w1_ref, y_ref, dinv_ref):
    deg = p_ref[0] + p_ref[1] + 1.0
    dinv = lax.rsqrt(deg)[:, None]
    xw = jnp.dot(x_ref[:], w1_ref[:], preferred_element_type=jnp.float32)
    y_ref[:] = dinv * xw
    dinv_ref[:] = dinv


_tc1_call = pl.pallas_call(
    _tc1_body,
    grid=(_GRID,),
    in_specs=[
        pl.BlockSpec((_NC, _BLK), lambda i: (0, i)),
        pl.BlockSpec((_BLK, _DIN), lambda i: (i, 0)),
        pl.BlockSpec((_DIN, _H), lambda i: (0, 0)),
    ],
    out_specs=[
        pl.BlockSpec((_BLK, _H), lambda i: (i, 0)),
        pl.BlockSpec((_BLK, 1), lambda i: (i, 0)),
    ],
    out_shape=[
        jax.ShapeDtypeStruct((_NPAD, _H), jnp.float32),
        jax.ShapeDtypeStruct((_NPAD, 1), jnp.float32),
    ],
)


def _tc2_body(p_ref, y1_ref, dinv_ref, b1_ref, w2_ref, y2_ref):
    s = p_ref[0] + p_ref[1] + y1_ref[:]
    dinv = dinv_ref[:]
    g1 = jnp.maximum(dinv * s + b1_ref[:], 0.0)
    y2_ref[:] = dinv * jnp.dot(g1, w2_ref[:], preferred_element_type=jnp.float32)


_tc2_call = pl.pallas_call(
    _tc2_body,
    grid=(_GRID,),
    in_specs=[
        pl.BlockSpec((_NC, _BLK, _H), lambda i: (0, i, 0)),
        pl.BlockSpec((_BLK, _H), lambda i: (i, 0)),
        pl.BlockSpec((_BLK, 1), lambda i: (i, 0)),
        pl.BlockSpec((1, _H), lambda i: (0, 0)),
        pl.BlockSpec((_H, _H), lambda i: (0, 0)),
    ],
    out_specs=pl.BlockSpec((_BLK, _H), lambda i: (i, 0)),
    out_shape=jax.ShapeDtypeStruct((_NPAD, _H), jnp.float32),
)


def _tc3_body(p_ref, y2_ref, dinv_ref, b2_ref, wd1_ref, bd1_ref, wd2_ref,
              bd2_ref, res_ref):
    s = p_ref[0] + p_ref[1] + y2_ref[:]
    dinv = dinv_ref[:]
    g2 = jnp.maximum(dinv * s + b2_ref[:], 0.0)
    d3 = jnp.maximum(
        jnp.dot(g2, wd1_ref[:], preferred_element_type=jnp.float32) + bd1_ref[:],
        0.0)
    res_ref[:] = jnp.dot(d3, wd2_ref[:],
                         preferred_element_type=jnp.float32) + bd2_ref[:]


_tc3_call = pl.pallas_call(
    _tc3_body,
    grid=(_GRID,),
    in_specs=[
        pl.BlockSpec((_NC, _BLK, _H), lambda i: (0, i, 0)),
        pl.BlockSpec((_BLK, _H), lambda i: (i, 0)),
        pl.BlockSpec((_BLK, 1), lambda i: (i, 0)),
        pl.BlockSpec((1, _H), lambda i: (0, 0)),
        pl.BlockSpec((_H, 32), lambda i: (0, 0)),
        pl.BlockSpec((1, 32), lambda i: (0, 0)),
        pl.BlockSpec((32, 1), lambda i: (0, 0)),
        pl.BlockSpec((1, 1), lambda i: (0, 0)),
    ],
    out_specs=pl.BlockSpec((_BLK, 1), lambda i: (i, 0)),
    out_shape=jax.ShapeDtypeStruct((_NPAD, 1), jnp.float32),
)


@jax.jit
def kernel(x, edge_index, W1, b1, W2, b2, Wd1, bd1, Wd2, bd2):
    src = edge_index[0]
    dst = edge_index[1]
    pad_idx = jnp.full((_EPAD - _E,), _N, jnp.int32)
    src_p = jnp.concatenate([src, pad_idx]).reshape(_NW, _KCH, _CHUNK)
    dst_p = jnp.concatenate([dst, pad_idx]).reshape(_NW, _KCH, _CHUNK)
    zeros64 = jnp.zeros((_NPAD, _H), jnp.float32)

    degp = _deg_call(dst_p.reshape(_NW, _NVEC, 16))
    y1, dinv = _tc1_call(degp, x, W1)
    s1 = _agg_call(y1, src_p, dst_p, zeros64)
    y2 = _tc2_call(s1, y1, dinv, b1.reshape(1, _H), W2)
    s2 = _agg_call(y2, src_p, dst_p, zeros64)
    res = _tc3_call(s2, y2, dinv, b2.reshape(1, _H), Wd1, bd1.reshape(1, 32),
                    Wd2, bd2.reshape(1, 1))
    return res[:_N]


# speedup vs baseline: 1.0553x; 1.0101x over previous
import functools

import jax
import jax.numpy as jnp
from jax import lax
from jax.experimental import pallas as pl
from jax.experimental.pallas import tpu as pltpu
from jax.experimental.pallas import tpu_sc as plsc

_N = 10000
_NPAD = 10240
_E = 320000
_DIN = 128
_H = 64
_NC = 2
_NS = 16
_NW = _NC * _NS
_CHUNK = 128
_KCH = 81
_NG = _KCH // 3
_EPW = _CHUNK * _KCH
_EPAD = _EPW * _NW
_RPT = _NPAD // _NS
_BLK = 1024
_GRID = _NPAD // _BLK

_sc_mesh = plsc.VectorSubcoreMesh(core_axis_name="c", subcore_axis_name="s")


_NVEC = _EPW // 16


def _deg_body(dst_hbm, out_hbm, dst_v, hist_v, red_v, out_v, part_sh, dsem):
    cid = lax.axis_index("c")
    sid = lax.axis_index("s")
    wid = cid * _NS + sid

    d = pltpu.async_copy(dst_hbm.at[wid], dst_v, dsem)

    def zero(i, c):
        hist_v[pl.ds(i * 16, 16)] = jnp.zeros((16,), jnp.float32)
        return c
    lax.fori_loop(0, _NPAD // 16, zero, 0)
    d.wait()

    ones = jnp.full((16,), 1.0, jnp.float32)

    def body(i, c):
        plsc.addupdate_scatter(hist_v, [dst_v[i]], ones)
        return c
    lax.fori_loop(0, _NVEC, body, 0)

    pltpu.sync_copy(hist_v, part_sh.at[sid])
    plsc.subcore_barrier()
    pltpu.sync_copy(part_sh.at[:, pl.ds(sid * _RPT, _RPT)], red_v)

    def red(i, c):
        acc = jnp.zeros((16,), jnp.float32)
        for r in range(_NS):
            acc = acc + red_v[r, pl.ds(i * 16, 16)]
        out_v[pl.ds(i * 16, 16)] = acc
        return c
    lax.fori_loop(0, _RPT // 16, red, 0)

    pltpu.sync_copy(out_v, out_hbm.at[cid, pl.ds(sid * _RPT, _RPT)])


_deg_call = pl.kernel(
    _deg_body,
    out_type=jax.ShapeDtypeStruct((_NC, _NPAD), jnp.float32),
    mesh=_sc_mesh,
    compiler_params=pltpu.CompilerParams(use_tc_tiling_on_sc=False,
                                         needs_layout_passes=False),
    scratch_types=[
        pltpu.VMEM((_NVEC, 16), jnp.int32),
        pltpu.VMEM((_NPAD,), jnp.float32),
        pltpu.VMEM((_NS, _RPT), jnp.float32),
        pltpu.VMEM((_RPT,), jnp.float32),
        pltpu.VMEM_SHARED((_NS, _NPAD), jnp.float32),
        pltpu.SemaphoreType.DMA,
    ],
)


def _agg_body(y_hbm, src_hbm, dst_hbm, zeros_hbm, out_hbm,
              src_v, dst_v, rows_v, rows_w, rows_u, y_sh, accum_sh, gsems,
              psems):
    cid = lax.axis_index("c")
    sid = lax.axis_index("s")
    wid = cid * _NS + sid

    pre = [
        pltpu.async_copy(zeros_hbm.at[pl.ds(sid * _RPT, _RPT)],
                         accum_sh.at[pl.ds(sid * _RPT, _RPT)], psems[0]),
        pltpu.async_copy(y_hbm.at[pl.ds(sid * _RPT, _RPT)],
                         y_sh.at[pl.ds(sid * _RPT, _RPT)], psems[1]),
        pltpu.async_copy(src_hbm.at[wid], src_v, psems[2]),
        pltpu.async_copy(dst_hbm.at[wid], dst_v, psems[3]),
    ]
    for d in pre:
        d.wait()
    plsc.subcore_barrier()

    bufs = [rows_v, rows_w, rows_u]

    def start_gather(buf, gs, j):
        pltpu.async_copy(y_sh.at[src_v.at[j]], buf, gs)

    def wait_gather(buf, gs):
        pltpu.make_async_copy(y_sh.at[src_v.at[0]], buf, gs).wait()

    for b in range(3):
        start_gather(bufs[b], gsems[b], b)

    def trio(i, c):
        base = i * 3
        for b in range(3):
            wait_gather(bufs[b], gsems[b])
            pltpu.sync_copy(bufs[b], accum_sh.at[dst_v.at[base + b]],
                            add=True)

            @pl.when(i < _NG - 1)
            def _():
                start_gather(bufs[b], gsems[b], base + 3 + b)
        return c
    lax.fori_loop(0, _NG, trio, 0)

    plsc.subcore_barrier()
    pltpu.sync_copy(accum_sh.at[pl.ds(sid * _RPT, _RPT)],
                    out_hbm.at[cid, pl.ds(sid * _RPT, _RPT)])


_agg_call = pl.kernel(
    _agg_body,
    out_type=jax.ShapeDtypeStruct((_NC, _NPAD, _H), jnp.float32),
    mesh=_sc_mesh,
    compiler_params=pltpu.CompilerParams(use_tc_tiling_on_sc=False),
    scratch_types=[
        pltpu.VMEM((_KCH, _CHUNK), jnp.int32),
        pltpu.VMEM((_KCH, _CHUNK), jnp.int32),
        pltpu.VMEM((_CHUNK, _H), jnp.float32),
        pltpu.VMEM((_CHUNK, _H), jnp.float32),
        pltpu.VMEM((_CHUNK, _H), jnp.float32),
        pltpu.VMEM_SHARED((_NPAD, _H), jnp.float32),
        pltpu.VMEM_SHARED((_NPAD, _H), jnp.float32),
        [pltpu.SemaphoreType.DMA] * 3,
        [pltpu.SemaphoreType.DMA] * 4,
    ],
)


def _dinv_of(p_ref):
    deg = p_ref[0] + p_ref[1] + 1.0
    return lax.rsqrt(deg)[:, None]


def _tc1_body(p_ref, x_ref, w1_ref, y_ref):
    xw = jnp.dot(x_ref[:], w1_ref[:], preferred_element_type=jnp.float32)
    y_ref[:] = _dinv_of(p_ref) * xw


_tc1_call = pl.pallas_call(
    _tc1_body,
    grid=(_GRID,),
    in_specs=[
        pl.BlockSpec((_NC, _BLK), lambda i: (0, i)),
        pl.BlockSpec((_BLK, _DIN), lambda i: (i, 0)),
        pl.BlockSpec((_DIN, _H), lambda i: (0, 0)),
    ],
    out_specs=pl.BlockSpec((_BLK, _H), lambda i: (i, 0)),
    out_shape=jax.ShapeDtypeStruct((_NPAD, _H), jnp.float32),
)


def _tc2_body(p_ref, s_ref, y1_ref, b1_ref, w2_ref, y2_ref):
    s = s_ref[0] + s_ref[1] + y1_ref[:]
    dinv = _dinv_of(p_ref)
    g1 = jnp.maximum(dinv * s + b1_ref[:], 0.0)
    y2_ref[:] = dinv * jnp.dot(g1, w2_ref[:], preferred_element_type=jnp.float32)


_tc2_call = pl.pallas_call(
    _tc2_body,
    grid=(_GRID,),
    in_specs=[
        pl.BlockSpec((_NC, _BLK), lambda i: (0, i)),
        pl.BlockSpec((_NC, _BLK, _H), lambda i: (0, i, 0)),
        pl.BlockSpec((_BLK, _H), lambda i: (i, 0)),
        pl.BlockSpec((1, _H), lambda i: (0, 0)),
        pl.BlockSpec((_H, _H), lambda i: (0, 0)),
    ],
    out_specs=pl.BlockSpec((_BLK, _H), lambda i: (i, 0)),
    out_shape=jax.ShapeDtypeStruct((_NPAD, _H), jnp.float32),
)


def _tc3_body(p_ref, s_ref, y2_ref, b2_ref, wd1_ref, bd1_ref, wd2_ref,
              bd2_ref, res_ref):
    s = s_ref[0] + s_ref[1] + y2_ref[:]
    dinv = _dinv_of(p_ref)
    g2 = jnp.maximum(dinv * s + b2_ref[:], 0.0)
    d3 = jnp.maximum(
        jnp.dot(g2, wd1_ref[:], preferred_element_type=jnp.float32) + bd1_ref[:],
        0.0)
    res_ref[:] = jnp.dot(d3, wd2_ref[:],
                         preferred_element_type=jnp.float32) + bd2_ref[:]


_tc3_call = pl.pallas_call(
    _tc3_body,
    grid=(_GRID,),
    in_specs=[
        pl.BlockSpec((_NC, _BLK), lambda i: (0, i)),
        pl.BlockSpec((_NC, _BLK, _H), lambda i: (0, i, 0)),
        pl.BlockSpec((_BLK, _H), lambda i: (i, 0)),
        pl.BlockSpec((1, _H), lambda i: (0, 0)),
        pl.BlockSpec((_H, 32), lambda i: (0, 0)),
        pl.BlockSpec((1, 32), lambda i: (0, 0)),
        pl.BlockSpec((32, 1), lambda i: (0, 0)),
        pl.BlockSpec((1, 1), lambda i: (0, 0)),
    ],
    out_specs=pl.BlockSpec((_BLK, 1), lambda i: (i, 0)),
    out_shape=jax.ShapeDtypeStruct((_NPAD, 1), jnp.float32),
)


@jax.jit
def kernel(x, edge_index, W1, b1, W2, b2, Wd1, bd1, Wd2, bd2):
    src = edge_index[0]
    dst = edge_index[1]
    pad_idx = jnp.full((_EPAD - _E,), _N, jnp.int32)
    src_p = jnp.concatenate([src, pad_idx]).reshape(_NW, _KCH, _CHUNK)
    dst_p = jnp.concatenate([dst, pad_idx]).reshape(_NW, _KCH, _CHUNK)
    zeros64 = jnp.zeros((_NPAD, _H), jnp.float32)

    degp = _deg_call(dst_p.reshape(_NW, _NVEC, 16))
    y1 = _tc1_call(degp, x, W1)
    s1 = _agg_call(y1, src_p, dst_p, zeros64)
    y2 = _tc2_call(degp, s1, y1, b1.reshape(1, _H), W2)
    s2 = _agg_call(y2, src_p, dst_p, zeros64)
    res = _tc3_call(degp, s2, y2, b2.reshape(1, _H), Wd1, bd1.reshape(1, 32),
                    Wd2, bd2.reshape(1, 1))
    return res[:_N]


# speedup vs baseline: 1.0863x; 1.0294x over previous
import functools

import jax
import jax.numpy as jnp
from jax import lax
from jax.experimental import pallas as pl
from jax.experimental.pallas import tpu as pltpu
from jax.experimental.pallas import tpu_sc as plsc

_N = 10000
_NPAD = 10240
_E = 320000
_DIN = 128
_H = 64
_NC = 2
_NS = 16
_NW = _NC * _NS
_CHUNK = 128
_KCH = 81
_NG = _KCH // 3
_EPW = _CHUNK * _KCH
_EPAD = _EPW * _NW
_RPT = _NPAD // _NS
_BLK = 2048
_GRID = _NPAD // _BLK

_sc_mesh = plsc.VectorSubcoreMesh(core_axis_name="c", subcore_axis_name="s")


_NVEC = _EPW // 16


def _deg_body(dst_hbm, out_hbm, dst_v, hist_v, red_v, out_v, part_sh, dsem):
    cid = lax.axis_index("c")
    sid = lax.axis_index("s")
    wid = cid * _NS + sid

    d = pltpu.async_copy(dst_hbm.at[wid], dst_v, dsem)

    def zero(i, c):
        hist_v[pl.ds(i * 16, 16)] = jnp.zeros((16,), jnp.float32)
        return c
    lax.fori_loop(0, _NPAD // 16, zero, 0)
    d.wait()

    ones = jnp.full((16,), 1.0, jnp.float32)

    def body(i, c):
        plsc.addupdate_scatter(hist_v, [dst_v[i]], ones)
        return c
    lax.fori_loop(0, _NVEC, body, 0)

    pltpu.sync_copy(hist_v, part_sh.at[sid])
    plsc.subcore_barrier()
    pltpu.sync_copy(part_sh.at[:, pl.ds(sid * _RPT, _RPT)], red_v)

    def red(i, c):
        acc = jnp.zeros((16,), jnp.float32)
        for r in range(_NS):
            acc = acc + red_v[r, pl.ds(i * 16, 16)]
        out_v[pl.ds(i * 16, 16)] = acc
        return c
    lax.fori_loop(0, _RPT // 16, red, 0)

    pltpu.sync_copy(out_v, out_hbm.at[cid, pl.ds(sid * _RPT, _RPT)])


_deg_call = pl.kernel(
    _deg_body,
    out_type=jax.ShapeDtypeStruct((_NC, _NPAD), jnp.float32),
    mesh=_sc_mesh,
    compiler_params=pltpu.CompilerParams(use_tc_tiling_on_sc=False,
                                         needs_layout_passes=False),
    scratch_types=[
        pltpu.VMEM((_NVEC, 16), jnp.int32),
        pltpu.VMEM((_NPAD,), jnp.float32),
        pltpu.VMEM((_NS, _RPT), jnp.float32),
        pltpu.VMEM((_RPT,), jnp.float32),
        pltpu.VMEM_SHARED((_NS, _NPAD), jnp.float32),
        pltpu.SemaphoreType.DMA,
    ],
)


def _agg_body(y_hbm, src_hbm, dst_hbm, zeros_hbm, out_hbm,
              src_v, dst_v, rows_v, rows_w, rows_u, y_sh, accum_sh, gsems,
              psems):
    cid = lax.axis_index("c")
    sid = lax.axis_index("s")
    wid = cid * _NS + sid

    pre = [
        pltpu.async_copy(zeros_hbm.at[pl.ds(sid * _RPT, _RPT)],
                         accum_sh.at[pl.ds(sid * _RPT, _RPT)], psems[0]),
        pltpu.async_copy(y_hbm.at[pl.ds(sid * _RPT, _RPT)],
                         y_sh.at[pl.ds(sid * _RPT, _RPT)], psems[1]),
        pltpu.async_copy(src_hbm.at[wid], src_v, psems[2]),
        pltpu.async_copy(dst_hbm.at[wid], dst_v, psems[3]),
    ]
    for d in pre:
        d.wait()
    plsc.subcore_barrier()

    bufs = [rows_v, rows_w, rows_u]

    def start_gather(buf, gs, j):
        pltpu.async_copy(y_sh.at[src_v.at[j]], buf, gs)

    def wait_gather(buf, gs):
        pltpu.make_async_copy(y_sh.at[src_v.at[0]], buf, gs).wait()

    for b in range(3):
        start_gather(bufs[b], gsems[b], b)

    def trio(i, c):
        base = i * 3
        for b in range(3):
            wait_gather(bufs[b], gsems[b])
            pltpu.sync_copy(bufs[b], accum_sh.at[dst_v.at[base + b]],
                            add=True)

            @pl.when(i < _NG - 1)
            def _():
                start_gather(bufs[b], gsems[b], base + 3 + b)
        return c
    lax.fori_loop(0, _NG, trio, 0)

    plsc.subcore_barrier()
    pltpu.sync_copy(accum_sh.at[pl.ds(sid * _RPT, _RPT)],
                    out_hbm.at[cid, pl.ds(sid * _RPT, _RPT)])


_agg_call = pl.kernel(
    _agg_body,
    out_type=jax.ShapeDtypeStruct((_NC, _NPAD, _H), jnp.float32),
    mesh=_sc_mesh,
    compiler_params=pltpu.CompilerParams(use_tc_tiling_on_sc=False),
    scratch_types=[
        pltpu.VMEM((_KCH, _CHUNK), jnp.int32),
        pltpu.VMEM((_KCH, _CHUNK), jnp.int32),
        pltpu.VMEM((_CHUNK, _H), jnp.float32),
        pltpu.VMEM((_CHUNK, _H), jnp.float32),
        pltpu.VMEM((_CHUNK, _H), jnp.float32),
        pltpu.VMEM_SHARED((_NPAD, _H), jnp.float32),
        pltpu.VMEM_SHARED((_NPAD, _H), jnp.float32),
        [pltpu.SemaphoreType.DMA] * 3,
        [pltpu.SemaphoreType.DMA] * 4,
    ],
)


def _dinv_of(p_ref):
    deg = p_ref[0] + p_ref[1] + 1.0
    return lax.rsqrt(deg)[:, None]


def _tc1_body(p_ref, x_ref, w1_ref, y_ref):
    xw = jnp.dot(x_ref[:], w1_ref[:], preferred_element_type=jnp.float32)
    y_ref[:] = _dinv_of(p_ref) * xw


_tc1_call = pl.pallas_call(
    _tc1_body,
    grid=(_GRID,),
    in_specs=[
        pl.BlockSpec((_NC, _BLK), lambda i: (0, i)),
        pl.BlockSpec((_BLK, _DIN), lambda i: (i, 0)),
        pl.BlockSpec((_DIN, _H), lambda i: (0, 0)),
    ],
    out_specs=pl.BlockSpec((_BLK, _H), lambda i: (i, 0)),
    out_shape=jax.ShapeDtypeStruct((_NPAD, _H), jnp.float32),
)


def _tc2_body(p_ref, s_ref, y1_ref, b1_ref, w2_ref, y2_ref):
    s = s_ref[0] + s_ref[1] + y1_ref[:]
    dinv = _dinv_of(p_ref)
    g1 = jnp.maximum(dinv * s + b1_ref[:], 0.0)
    y2_ref[:] = dinv * jnp.dot(g1, w2_ref[:], preferred_element_type=jnp.float32)


_tc2_call = pl.pallas_call(
    _tc2_body,
    grid=(_GRID,),
    in_specs=[
        pl.BlockSpec((_NC, _BLK), lambda i: (0, i)),
        pl.BlockSpec((_NC, _BLK, _H), lambda i: (0, i, 0)),
        pl.BlockSpec((_BLK, _H), lambda i: (i, 0)),
        pl.BlockSpec((1, _H), lambda i: (0, 0)),
        pl.BlockSpec((_H, _H), lambda i: (0, 0)),
    ],
    out_specs=pl.BlockSpec((_BLK, _H), lambda i: (i, 0)),
    out_shape=jax.ShapeDtypeStruct((_NPAD, _H), jnp.float32),
)


def _tc3_body(p_ref, s_ref, y2_ref, b2_ref, wd1_ref, bd1_ref, wd2_ref,
              bd2_ref, res_ref):
    s = s_ref[0] + s_ref[1] + y2_ref[:]
    dinv = _dinv_of(p_ref)
    g2 = jnp.maximum(dinv * s + b2_ref[:], 0.0)
    d3 = jnp.maximum(
        jnp.dot(g2, wd1_ref[:], preferred_element_type=jnp.float32) + bd1_ref[:],
        0.0)
    res_ref[:] = jnp.dot(d3, wd2_ref[:],
                         preferred_element_type=jnp.float32) + bd2_ref[:]


_tc3_call = pl.pallas_call(
    _tc3_body,
    grid=(_GRID,),
    in_specs=[
        pl.BlockSpec((_NC, _BLK), lambda i: (0, i)),
        pl.BlockSpec((_NC, _BLK, _H), lambda i: (0, i, 0)),
        pl.BlockSpec((_BLK, _H), lambda i: (i, 0)),
        pl.BlockSpec((1, _H), lambda i: (0, 0)),
        pl.BlockSpec((_H, 32), lambda i: (0, 0)),
        pl.BlockSpec((1, 32), lambda i: (0, 0)),
        pl.BlockSpec((32, 1), lambda i: (0, 0)),
        pl.BlockSpec((1, 1), lambda i: (0, 0)),
    ],
    out_specs=pl.BlockSpec((_BLK, 1), lambda i: (i, 0)),
    out_shape=jax.ShapeDtypeStruct((_N, 1), jnp.float32),
)


@jax.jit
def kernel(x, edge_index, W1, b1, W2, b2, Wd1, bd1, Wd2, bd2):
    src = edge_index[0]
    dst = edge_index[1]
    pad_idx = jnp.full((_EPAD - _E,), _N, jnp.int32)
    src_p = jnp.concatenate([src, pad_idx]).reshape(_NW, _KCH, _CHUNK)
    dst_p = jnp.concatenate([dst, pad_idx]).reshape(_NW, _KCH, _CHUNK)
    zeros64 = jnp.zeros((_NPAD, _H), jnp.float32)

    degp = _deg_call(dst_p.reshape(_NW, _NVEC, 16))
    y1 = _tc1_call(degp, x, W1)
    s1 = _agg_call(y1, src_p, dst_p, zeros64)
    y2 = _tc2_call(degp, s1, y1, b1.reshape(1, _H), W2)
    s2 = _agg_call(y2, src_p, dst_p, zeros64)
    res = _tc3_call(degp, s2, y2, b2.reshape(1, _H), Wd1, bd1.reshape(1, 32),
                    Wd2, bd2.reshape(1, 1))
    return res


# speedup vs baseline: 1.0868x; 1.0004x over previous
import jax
import jax.numpy as jnp
from jax import lax
from jax.experimental import pallas as pl
from jax.experimental.pallas import tpu as pltpu
from jax.experimental.pallas import tpu_sc as plsc

_N = 10000
_NPAD = 10240
_E = 320000
_DIN = 128
_H = 64
_NC = 2
_NS = 16
_NW = _NC * _NS
_CHUNK = 128
_KCH = 81
_NG = _KCH // 3
_EPW = _CHUNK * _KCH
_EPAD = _EPW * _NW
_RPT = _NPAD // _NS
_BLK = 2048
_GRID = _NPAD // _BLK

_sc_mesh = plsc.VectorSubcoreMesh(core_axis_name="c", subcore_axis_name="s")


_NVEC = _EPW // 16


def _deg_body(dst_hbm, out_hbm, dst_v, hist_v, red_v, out_v, part_sh, dsem):
    cid = lax.axis_index("c")
    sid = lax.axis_index("s")
    wid = cid * _NS + sid

    d = pltpu.async_copy(dst_hbm.at[wid], dst_v, dsem)

    def zero(i, c):
        hist_v[pl.ds(i * 16, 16)] = jnp.zeros((16,), jnp.float32)
        return c
    lax.fori_loop(0, _NPAD // 16, zero, 0)
    d.wait()

    ones = jnp.full((16,), 1.0, jnp.float32)

    def body(i, c):
        plsc.addupdate_scatter(hist_v, [dst_v[i]], ones)
        return c
    lax.fori_loop(0, _NVEC, body, 0)

    pltpu.sync_copy(hist_v, part_sh.at[sid])
    plsc.subcore_barrier()
    pltpu.sync_copy(part_sh.at[:, pl.ds(sid * _RPT, _RPT)], red_v)

    def red(i, c):
        acc = jnp.zeros((16,), jnp.float32)
        for r in range(_NS):
            acc = acc + red_v[r, pl.ds(i * 16, 16)]
        out_v[pl.ds(i * 16, 16)] = acc
        return c
    lax.fori_loop(0, _RPT // 16, red, 0)

    pltpu.sync_copy(out_v, out_hbm.at[cid, pl.ds(sid * _RPT, _RPT)])


_deg_call = pl.kernel(
    _deg_body,
    out_type=jax.ShapeDtypeStruct((_NC, _NPAD), jnp.float32),
    mesh=_sc_mesh,
    compiler_params=pltpu.CompilerParams(use_tc_tiling_on_sc=False,
                                         needs_layout_passes=False),
    scratch_types=[
        pltpu.VMEM((_NVEC, 16), jnp.int32),
        pltpu.VMEM((_NPAD,), jnp.float32),
        pltpu.VMEM((_NS, _RPT), jnp.float32),
        pltpu.VMEM((_RPT,), jnp.float32),
        pltpu.VMEM_SHARED((_NS, _NPAD), jnp.float32),
        pltpu.SemaphoreType.DMA,
    ],
)


def _agg_body(y_hbm, src_hbm, dst_hbm, zeros_hbm, out_hbm,
              src_v, dst_v, rows_v, rows_w, rows_u, y_sh, accum_sh, gsems,
              psems):
    cid = lax.axis_index("c")
    sid = lax.axis_index("s")
    wid = cid * _NS + sid

    pre = [
        pltpu.async_copy(zeros_hbm.at[pl.ds(sid * _RPT, _RPT)],
                         accum_sh.at[pl.ds(sid * _RPT, _RPT)], psems[0]),
        pltpu.async_copy(y_hbm.at[pl.ds(sid * _RPT, _RPT)],
                         y_sh.at[pl.ds(sid * _RPT, _RPT)], psems[1]),
        pltpu.async_copy(src_hbm.at[wid], src_v, psems[2]),
        pltpu.async_copy(dst_hbm.at[wid], dst_v, psems[3]),
    ]
    for d in pre:
        d.wait()
    plsc.subcore_barrier()

    bufs = [rows_v, rows_w, rows_u]

    def start_gather(buf, gs, j):
        pltpu.async_copy(y_sh.at[src_v.at[j]], buf, gs)

    def wait_gather(buf, gs):
        pltpu.make_async_copy(y_sh.at[src_v.at[0]], buf, gs).wait()

    for b in range(3):
        start_gather(bufs[b], gsems[b], b)

    def trio(i, c):
        base = i * 3
        for b in range(3):
            wait_gather(bufs[b], gsems[b])
            pltpu.sync_copy(bufs[b], accum_sh.at[dst_v.at[base + b]],
                            add=True)

            @pl.when(i < _NG - 1)
            def _():
                start_gather(bufs[b], gsems[b], base + 3 + b)
        return c
    lax.fori_loop(0, _NG, trio, 0)

    plsc.subcore_barrier()
    pltpu.sync_copy(accum_sh.at[pl.ds(sid * _RPT, _RPT)],
                    out_hbm.at[cid, pl.ds(sid * _RPT, _RPT)])


_agg_call = pl.kernel(
    _agg_body,
    out_type=jax.ShapeDtypeStruct((_NC, _NPAD, _H), jnp.float32),
    mesh=_sc_mesh,
    compiler_params=pltpu.CompilerParams(use_tc_tiling_on_sc=False),
    scratch_types=[
        pltpu.VMEM((_KCH, _CHUNK), jnp.int32),
        pltpu.VMEM((_KCH, _CHUNK), jnp.int32),
        pltpu.VMEM((_CHUNK, _H), jnp.float32),
        pltpu.VMEM((_CHUNK, _H), jnp.float32),
        pltpu.VMEM((_CHUNK, _H), jnp.float32),
        pltpu.VMEM_SHARED((_NPAD, _H), jnp.float32),
        pltpu.VMEM_SHARED((_NPAD, _H), jnp.float32),
        [pltpu.SemaphoreType.DMA] * 3,
        [pltpu.SemaphoreType.DMA] * 4,
    ],
)


def _dinv_of(p_ref):
    deg = p_ref[0] + p_ref[1] + 1.0
    return lax.rsqrt(deg)[:, None]


def _tc1_body(p_ref, x_ref, w1_ref, y_ref):
    xw = jnp.dot(x_ref[:], w1_ref[:], preferred_element_type=jnp.float32)
    y_ref[:] = _dinv_of(p_ref) * xw


_tc1_call = pl.pallas_call(
    _tc1_body,
    grid=(_GRID,),
    in_specs=[
        pl.BlockSpec((_NC, _BLK), lambda i: (0, i)),
        pl.BlockSpec((_BLK, _DIN), lambda i: (i, 0)),
        pl.BlockSpec((_DIN, _H), lambda i: (0, 0)),
    ],
    out_specs=pl.BlockSpec((_BLK, _H), lambda i: (i, 0)),
    out_shape=jax.ShapeDtypeStruct((_NPAD, _H), jnp.float32),
)


def _tc2_body(p_ref, s_ref, y1_ref, b1_ref, w2_ref, y2_ref):
    s = s_ref[0] + s_ref[1] + y1_ref[:]
    dinv = _dinv_of(p_ref)
    g1 = jnp.maximum(dinv * s + b1_ref[:], 0.0)
    y2_ref[:] = dinv * jnp.dot(g1, w2_ref[:], preferred_element_type=jnp.float32)


_tc2_call = pl.pallas_call(
    _tc2_body,
    grid=(_GRID,),
    in_specs=[
        pl.BlockSpec((_NC, _BLK), lambda i: (0, i)),
        pl.BlockSpec((_NC, _BLK, _H), lambda i: (0, i, 0)),
        pl.BlockSpec((_BLK, _H), lambda i: (i, 0)),
        pl.BlockSpec((1, _H), lambda i: (0, 0)),
        pl.BlockSpec((_H, _H), lambda i: (0, 0)),
    ],
    out_specs=pl.BlockSpec((_BLK, _H), lambda i: (i, 0)),
    out_shape=jax.ShapeDtypeStruct((_NPAD, _H), jnp.float32),
)


def _tc3_body(p_ref, s_ref, y2_ref, b2_ref, wd1_ref, bd1_ref, wd2_ref,
              bd2_ref, res_ref):
    s = s_ref[0] + s_ref[1] + y2_ref[:]
    dinv = _dinv_of(p_ref)
    g2 = jnp.maximum(dinv * s + b2_ref[:], 0.0)
    d3 = jnp.maximum(
        jnp.dot(g2, wd1_ref[:], preferred_element_type=jnp.float32) + bd1_ref[:],
        0.0)
    res_ref[:] = jnp.dot(d3, wd2_ref[:],
                         preferred_element_type=jnp.float32) + bd2_ref[:]


_tc3_call = pl.pallas_call(
    _tc3_body,
    grid=(_GRID,),
    in_specs=[
        pl.BlockSpec((_NC, _BLK), lambda i: (0, i)),
        pl.BlockSpec((_NC, _BLK, _H), lambda i: (0, i, 0)),
        pl.BlockSpec((_BLK, _H), lambda i: (i, 0)),
        pl.BlockSpec((1, _H), lambda i: (0, 0)),
        pl.BlockSpec((_H, 32), lambda i: (0, 0)),
        pl.BlockSpec((1, 32), lambda i: (0, 0)),
        pl.BlockSpec((32, 1), lambda i: (0, 0)),
        pl.BlockSpec((1, 1), lambda i: (0, 0)),
    ],
    out_specs=pl.BlockSpec((_BLK, 1), lambda i: (i, 0)),
    out_shape=jax.ShapeDtypeStruct((_N, 1), jnp.float32),
)


@jax.jit
def kernel(x, edge_index, W1, b1, W2, b2, Wd1, bd1, Wd2, bd2):
    src = edge_index[0]
    dst = edge_index[1]
    pad_idx = jnp.full((_EPAD - _E,), _N, jnp.int32)
    src_p = jnp.concatenate([src, pad_idx]).reshape(_NW, _KCH, _CHUNK)
    dst_p = jnp.concatenate([dst, pad_idx]).reshape(_NW, _KCH, _CHUNK)
    zeros64 = jnp.zeros((_NPAD, _H), jnp.float32)

    degp = _deg_call(dst_p.reshape(_NW, _NVEC, 16))
    y1 = _tc1_call(degp, x, W1)
    s1 = _agg_call(y1, src_p, dst_p, zeros64)
    y2 = _tc2_call(degp, s1, y1, b1.reshape(1, _H), W2)
    s2 = _agg_call(y2, src_p, dst_p, zeros64)
    res = _tc3_call(degp, s2, y2, b2.reshape(1, _H), Wd1, bd1.reshape(1, 32),
                    Wd2, bd2.reshape(1, 1))
    return res
